# baseline (speedup 1.0000x reference)
import functools

import jax
import jax.numpy as jnp
from jax.experimental import pallas as pl
from jax.experimental.pallas import tpu as pltpu

N = 10000
D = 128
G = 512
TR = 400
NSTEP = N // TR


def _xw_kernel(x_ref, w_ref, out_ref):
    out_ref[...] = jnp.dot(x_ref[...], w_ref[...],
                           preferred_element_type=jnp.float32)


def _phase0_kernel(adj_ref, z0_ref, b0_ref, w1_ref, z1hi_ref, z1lo_ref,
                   mask_ref):
    adj = adj_ref[...]
    acc = jnp.dot(adj, z0_ref[...], preferred_element_type=jnp.float32)
    h = jnp.maximum(acc + b0_ref[...], 0.0)
    z1 = jnp.dot(h, w1_ref[...], preferred_element_type=jnp.float32)
    hi = z1.astype(jnp.float8_e4m3fn)
    z1hi_ref[...] = hi
    z1lo_ref[...] = (z1 - hi.astype(jnp.float32)).astype(jnp.float8_e4m3fn)
    mask_ref[...] = adj.astype(jnp.float8_e4m3fn)


def _phase1_kernel(mask_ref, z1hi_ref, z1lo_ref, idx_ref, b1_ref,
                   g1_ref, be1_ref, g2_ref, be2_ref, fw_ref, fb_ref,
                   out_ref, pool_scr, cnt_scr, s1_scr, s2_scr):
    i = pl.program_id(0)

    @pl.when(i == 0)
    def _():
        pool_scr[...] = jnp.zeros((G, D), jnp.float32)
        cnt_scr[...] = jnp.zeros((G, TR), jnp.float32)
        s1_scr[...] = jnp.zeros((1, D), jnp.float32)
        s2_scr[...] = jnp.zeros((1, D), jnp.float32)

    adj = mask_ref[...]
    acc = (jnp.dot(adj, z1hi_ref[...], preferred_element_type=jnp.float32)
           + jnp.dot(adj, z1lo_ref[...], preferred_element_type=jnp.float32))
    h2 = jnp.maximum(acc + b1_ref[...], 0.0)
    s1_scr[...] += jnp.sum(h2, axis=0, keepdims=True)
    s2_scr[...] += jnp.sum(h2 * h2, axis=0, keepdims=True)
    ids = idx_ref[0, :, :]
    gi = jax.lax.broadcasted_iota(jnp.int32, (G, TR), 0)
    onehot = (gi == ids).astype(jnp.float32)
    pool_scr[...] += jnp.dot(onehot, h2, preferred_element_type=jnp.float32)
    cnt_scr[...] += onehot

    @pl.when(i == NSTEP - 1)
    def _():
        n_f = jnp.float32(N)
        mean1 = s1_scr[...] / n_f
        var1 = s2_scr[...] / n_f - mean1 * mean1
        a1 = g1_ref[...] / jnp.sqrt(var1 + 1e-5)
        c1 = be1_ref[...] - mean1 * a1
        cnt = jnp.sum(cnt_scr[...], axis=1, keepdims=True)
        pooled = pool_scr[...] * a1 + cnt * c1
        mean2 = jnp.mean(pooled, axis=0, keepdims=True)
        var2 = jnp.mean((pooled - mean2) ** 2, axis=0, keepdims=True)
        y = (pooled - mean2) / jnp.sqrt(var2 + 1e-5) * g2_ref[...] + be2_ref[...]
        out = jnp.dot(y, fw_ref[...], preferred_element_type=jnp.float32)
        out_ref[...] = jnp.maximum(out + fb_ref[...], 0.0)


def _const(shape):
    return pl.BlockSpec(shape, lambda i: tuple(0 for _ in shape))


@functools.partial(jax.jit, static_argnames=("interpret",))
def _run(adj, x, idx, W0, b0, W1, b1, gamma1, beta1, gamma2, beta2,
         fc1_W, fc1_b, interpret=False):
    f32 = jnp.float32
    z0 = pl.pallas_call(
        _xw_kernel,
        out_shape=jax.ShapeDtypeStruct((N, D), f32),
        interpret=interpret,
    )(x, W0)

    row = pl.BlockSpec((TR, N), lambda i: (i, 0))
    outrow = pl.BlockSpec((TR, D), lambda i: (i, 0))
    f8 = jnp.float8_e4m3fn
    z1hi, z1lo, mask = pl.pallas_call(
        _phase0_kernel,
        grid=(NSTEP,),
        in_specs=[row, _const((N, D)), _const((1, D)), _const((D, D))],
        out_specs=[outrow, outrow, row],
        out_shape=[jax.ShapeDtypeStruct((N, D), f8),
                   jax.ShapeDtypeStruct((N, D), f8),
                   jax.ShapeDtypeStruct((N, N), f8)],
        interpret=interpret,
    )(adj, z0, b0.reshape(1, D), W1)

    idx_spec = pl.BlockSpec((1, 1, TR), lambda i: (i, 0, 0))
    out = pl.pallas_call(
        _phase1_kernel,
        grid=(NSTEP,),
        in_specs=[row, _const((N, D)), _const((N, D)), idx_spec,
                  _const((1, D)), _const((1, D)), _const((1, D)),
                  _const((1, D)), _const((1, D)), _const((D, D)),
                  _const((1, D))],
        out_specs=_const((G, D)),
        out_shape=jax.ShapeDtypeStruct((G, D), f32),
        scratch_shapes=[pltpu.VMEM((G, D), f32), pltpu.VMEM((G, TR), f32),
                        pltpu.VMEM((1, D), f32), pltpu.VMEM((1, D), f32)],
        interpret=interpret,
    )(mask, z1hi, z1lo, idx.reshape(NSTEP, 1, TR).astype(jnp.int32),
      b1.reshape(1, D), gamma1.reshape(1, D), beta1.reshape(1, D),
      gamma2.reshape(1, D), beta2.reshape(1, D), fc1_W, fc1_b.reshape(1, D))
    return out


def kernel(adj, final_features, segment, idx, W0, b0, W1, b1,
           gamma1, beta1, gamma2, beta2, fc1_W, fc1_b):
    return _run(adj, final_features, idx, W0, b0, W1, b1,
                gamma1, beta1, gamma2, beta2, fc1_W, fc1_b)


# speedup vs baseline: 1.0860x; 1.0860x over previous
import functools

import jax
import jax.numpy as jnp
from jax.experimental import pallas as pl
from jax.experimental.pallas import tpu as pltpu

N = 10000
D = 128
G = 512
TR = 400
NSTEP = N // TR


def _xw_kernel(x_ref, w_ref, out_ref):
    out_ref[...] = jnp.dot(x_ref[...], w_ref[...],
                           preferred_element_type=jnp.float32)


def _phase0_kernel(adj_ref, z0_ref, b0_ref, w1_ref, z1cat_ref, mask_ref):
    adj = adj_ref[...]
    acc = jnp.dot(adj, z0_ref[...], preferred_element_type=jnp.float32)
    h = jnp.maximum(acc + b0_ref[...], 0.0)
    z1 = jnp.dot(h, w1_ref[...], preferred_element_type=jnp.float32)
    hi = z1.astype(jnp.float8_e4m3fn)
    lo = (z1 - hi.astype(jnp.float32)).astype(jnp.float8_e4m3fn)
    z1cat_ref[...] = jnp.concatenate([hi, lo], axis=1)
    mask_ref[...] = adj.astype(jnp.float8_e4m3fn)


def _phase1_kernel(mask_ref, z1cat_ref, idx_ref, b1_ref,
                   g1_ref, be1_ref, g2_ref, be2_ref, fw_ref, fb_ref,
                   out_ref, pool_scr, cnt_scr, s1_scr, s2_scr):
    i = pl.program_id(0)

    @pl.when(i == 0)
    def _():
        pool_scr[...] = jnp.zeros((G, D), jnp.float32)
        cnt_scr[...] = jnp.zeros((G, TR), jnp.float32)
        s1_scr[...] = jnp.zeros((1, D), jnp.float32)
        s2_scr[...] = jnp.zeros((1, D), jnp.float32)

    adj = mask_ref[...]
    r = jnp.dot(adj, z1cat_ref[...], preferred_element_type=jnp.float32)
    acc = r[:, :D] + r[:, D:]
    h2 = jnp.maximum(acc + b1_ref[...], 0.0)
    s1_scr[...] += jnp.sum(h2, axis=0, keepdims=True)
    s2_scr[...] += jnp.sum(h2 * h2, axis=0, keepdims=True)
    ids = idx_ref[0, :, :]
    gi = jax.lax.broadcasted_iota(jnp.int32, (G, TR), 0)
    onehot = (gi == ids).astype(jnp.float32)
    pool_scr[...] += jnp.dot(onehot, h2, preferred_element_type=jnp.float32)
    cnt_scr[...] += onehot

    @pl.when(i == NSTEP - 1)
    def _():
        n_f = jnp.float32(N)
        mean1 = s1_scr[...] / n_f
        var1 = s2_scr[...] / n_f - mean1 * mean1
        a1 = g1_ref[...] / jnp.sqrt(var1 + 1e-5)
        c1 = be1_ref[...] - mean1 * a1
        cnt = jnp.sum(cnt_scr[...], axis=1, keepdims=True)
        pooled = pool_scr[...] * a1 + cnt * c1
        mean2 = jnp.mean(pooled, axis=0, keepdims=True)
        var2 = jnp.mean((pooled - mean2) ** 2, axis=0, keepdims=True)
        y = (pooled - mean2) / jnp.sqrt(var2 + 1e-5) * g2_ref[...] + be2_ref[...]
        out = jnp.dot(y, fw_ref[...], preferred_element_type=jnp.float32)
        out_ref[...] = jnp.maximum(out + fb_ref[...], 0.0)


def _const(shape):
    return pl.BlockSpec(shape, lambda i: tuple(0 for _ in shape))


@functools.partial(jax.jit, static_argnames=("interpret",))
def _run(adj, x, idx, W0, b0, W1, b1, gamma1, beta1, gamma2, beta2,
         fc1_W, fc1_b, interpret=False):
    f32 = jnp.float32
    z0 = pl.pallas_call(
        _xw_kernel,
        out_shape=jax.ShapeDtypeStruct((N, D), f32),
        interpret=interpret,
    )(x, W0)

    row = pl.BlockSpec((TR, N), lambda i: (i, 0))
    outrow = pl.BlockSpec((TR, D), lambda i: (i, 0))
    f8 = jnp.float8_e4m3fn
    outcat = pl.BlockSpec((TR, 2 * D), lambda i: (i, 0))
    z1cat, mask = pl.pallas_call(
        _phase0_kernel,
        grid=(NSTEP,),
        in_specs=[row, _const((N, D)), _const((1, D)), _const((D, D))],
        out_specs=[outcat, row],
        out_shape=[jax.ShapeDtypeStruct((N, 2 * D), f8),
                   jax.ShapeDtypeStruct((N, N), f8)],
        interpret=interpret,
    )(adj, z0, b0.reshape(1, D), W1)

    idx_spec = pl.BlockSpec((1, 1, TR), lambda i: (i, 0, 0))
    out = pl.pallas_call(
        _phase1_kernel,
        grid=(NSTEP,),
        in_specs=[row, _const((N, 2 * D)), idx_spec,
                  _const((1, D)), _const((1, D)), _const((1, D)),
                  _const((1, D)), _const((1, D)), _const((D, D)),
                  _const((1, D))],
        out_specs=_const((G, D)),
        out_shape=jax.ShapeDtypeStruct((G, D), f32),
        scratch_shapes=[pltpu.VMEM((G, D), f32), pltpu.VMEM((G, TR), f32),
                        pltpu.VMEM((1, D), f32), pltpu.VMEM((1, D), f32)],
        interpret=interpret,
    )(mask, z1cat, idx.reshape(NSTEP, 1, TR).astype(jnp.int32),
      b1.reshape(1, D), gamma1.reshape(1, D), beta1.reshape(1, D),
      gamma2.reshape(1, D), beta2.reshape(1, D), fc1_W, fc1_b.reshape(1, D))
    return out


def kernel(adj, final_features, segment, idx, W0, b0, W1, b1,
           gamma1, beta1, gamma2, beta2, fc1_W, fc1_b):
    return _run(adj, final_features, idx, W0, b0, W1, b1,
                gamma1, beta1, gamma2, beta2, fc1_W, fc1_b)


# speedup vs baseline: 1.1134x; 1.0253x over previous
import functools

import jax
import jax.numpy as jnp
from jax.experimental import pallas as pl
from jax.experimental.pallas import tpu as pltpu

N = 10000
D = 128
G = 512
TR = 400
NSTEP = N // TR


def _phase0_kernel(adj_ref, x_ref, w0_ref, b0_ref, w1_ref, z1_ref, mask_ref):
    adj = adj_ref[...]
    ax = jnp.dot(adj, x_ref[...], preferred_element_type=jnp.float32)
    h = jnp.maximum(jnp.dot(ax, w0_ref[...],
                            preferred_element_type=jnp.float32)
                    + b0_ref[...], 0.0)
    z1 = jnp.dot(h, w1_ref[...], preferred_element_type=jnp.float32)
    hi = z1.astype(jnp.float8_e4m3fn)
    lo = (z1 - hi.astype(jnp.float32)).astype(jnp.float8_e4m3fn)
    z1_ref[...] = jnp.concatenate([hi, lo], axis=1)
    mask_ref[...] = adj.astype(jnp.float8_e4m3fn)


def _phase1_kernel(mask_ref, z1cat_ref, idx_ref, b1_ref,
                   g1_ref, be1_ref, g2_ref, be2_ref, fw_ref, fb_ref,
                   out_ref, pool_scr, cnt_scr, s1_scr, s2_scr):
    i = pl.program_id(0)

    @pl.when(i == 0)
    def _():
        pool_scr[...] = jnp.zeros((G, D), jnp.float32)
        cnt_scr[...] = jnp.zeros((G, TR), jnp.float32)
        s1_scr[...] = jnp.zeros((1, D), jnp.float32)
        s2_scr[...] = jnp.zeros((1, D), jnp.float32)

    adj = mask_ref[...]
    r = jnp.dot(adj, z1cat_ref[...], preferred_element_type=jnp.float32)
    acc = r[:, :D] + r[:, D:]
    h2 = jnp.maximum(acc + b1_ref[...], 0.0)
    s1_scr[...] += jnp.sum(h2, axis=0, keepdims=True)
    s2_scr[...] += jnp.sum(h2 * h2, axis=0, keepdims=True)
    ids = idx_ref[0, :, :]
    gi = jax.lax.broadcasted_iota(jnp.int32, (G, TR), 0)
    onehot = (gi == ids).astype(jnp.float32)
    pool_scr[...] += jnp.dot(onehot, h2, preferred_element_type=jnp.float32)
    cnt_scr[...] += onehot

    @pl.when(i == NSTEP - 1)
    def _():
        n_f = jnp.float32(N)
        mean1 = s1_scr[...] / n_f
        var1 = s2_scr[...] / n_f - mean1 * mean1
        a1 = g1_ref[...] / jnp.sqrt(var1 + 1e-5)
        c1 = be1_ref[...] - mean1 * a1
        cnt = jnp.sum(cnt_scr[...], axis=1, keepdims=True)
        pooled = pool_scr[...] * a1 + cnt * c1
        mean2 = jnp.mean(pooled, axis=0, keepdims=True)
        var2 = jnp.mean((pooled - mean2) ** 2, axis=0, keepdims=True)
        y = (pooled - mean2) / jnp.sqrt(var2 + 1e-5) * g2_ref[...] + be2_ref[...]
        out = jnp.dot(y, fw_ref[...], preferred_element_type=jnp.float32)
        out_ref[...] = jnp.maximum(out + fb_ref[...], 0.0)


def _const(shape):
    return pl.BlockSpec(shape, lambda i: tuple(0 for _ in shape))


@functools.partial(jax.jit, static_argnames=("interpret",))
def _run(adj, x, idx, W0, b0, W1, b1, gamma1, beta1, gamma2, beta2,
         fc1_W, fc1_b, interpret=False):
    f32 = jnp.float32
    row = pl.BlockSpec((TR, N), lambda i: (i, 0))
    outrow = pl.BlockSpec((TR, D), lambda i: (i, 0))
    f8 = jnp.float8_e4m3fn
    z1, mask = pl.pallas_call(
        _phase0_kernel,
        grid=(NSTEP,),
        in_specs=[row, _const((N, D)), _const((D, D)), _const((1, D)),
                  _const((D, D))],
        out_specs=[pl.BlockSpec((TR, 2 * D), lambda i: (i, 0)), row],
        out_shape=[jax.ShapeDtypeStruct((N, 2 * D), f8),
                   jax.ShapeDtypeStruct((N, N), f8)],
        interpret=interpret,
    )(adj, x, W0, b0.reshape(1, D), W1)

    idx_spec = pl.BlockSpec((1, 1, TR), lambda i: (i, 0, 0))
    out = pl.pallas_call(
        _phase1_kernel,
        grid=(NSTEP,),
        in_specs=[row, _const((N, 2 * D)), idx_spec,
                  _const((1, D)), _const((1, D)), _const((1, D)),
                  _const((1, D)), _const((1, D)), _const((D, D)),
                  _const((1, D))],
        out_specs=_const((G, D)),
        out_shape=jax.ShapeDtypeStruct((G, D), f32),
        scratch_shapes=[pltpu.VMEM((G, D), f32), pltpu.VMEM((G, TR), f32),
                        pltpu.VMEM((1, D), f32), pltpu.VMEM((1, D), f32)],
        interpret=interpret,
    )(mask, z1, idx.reshape(NSTEP, 1, TR).astype(jnp.int32),
      b1.reshape(1, D), gamma1.reshape(1, D), beta1.reshape(1, D),
      gamma2.reshape(1, D), beta2.reshape(1, D), fc1_W, fc1_b.reshape(1, D))
    return out


def kernel(adj, final_features, segment, idx, W0, b0, W1, b1,
           gamma1, beta1, gamma2, beta2, fc1_W, fc1_b):
    return _run(adj, final_features, idx, W0, b0, W1, b1,
                gamma1, beta1, gamma2, beta2, fc1_W, fc1_b)


# speedup vs baseline: 1.1340x; 1.0185x over previous
import functools

import jax
import jax.numpy as jnp
from jax.experimental import pallas as pl
from jax.experimental.pallas import tpu as pltpu

N = 10000
D = 128
G = 512
TR = 400
NSTEP = N // TR


def _phase0_kernel(adj_ref, x_ref, w0_ref, b0_ref, w1_ref, z1_ref, mask_ref):
    adj = adj_ref[...]
    ax = jnp.dot(adj, x_ref[...], preferred_element_type=jnp.float32)
    h = jnp.maximum(jnp.dot(ax, w0_ref[...],
                            preferred_element_type=jnp.float32,
                            precision=jax.lax.Precision.HIGHEST)
                    + b0_ref[...], 0.0)
    z1 = jnp.dot(h, w1_ref[...], preferred_element_type=jnp.float32,
                 precision=jax.lax.Precision.HIGHEST)
    hi = z1.astype(jnp.float8_e4m3fn)
    lo = (z1 - hi.astype(jnp.float32)).astype(jnp.float8_e4m3fn)
    z1_ref[...] = jnp.concatenate([hi, lo], axis=1)
    mask_ref[...] = adj.astype(jnp.float8_e4m3fn)


def _phase1_kernel(mask_ref, z1cat_ref, idx_ref, b1_ref,
                   g1_ref, be1_ref, g2_ref, be2_ref, fw_ref, fb_ref,
                   out_ref, pool_scr, cnt_scr, s1_scr, s2_scr):
    i = pl.program_id(0)

    @pl.when(i == 0)
    def _():
        pool_scr[...] = jnp.zeros((G, D), jnp.float32)
        cnt_scr[...] = jnp.zeros((G, TR), jnp.float32)
        s1_scr[...] = jnp.zeros((1, D), jnp.float32)
        s2_scr[...] = jnp.zeros((1, D), jnp.float32)

    adj = mask_ref[...]
    r = jnp.dot(adj, z1cat_ref[...], preferred_element_type=jnp.float32)
    acc = r[:, :D] + r[:, D:]
    h2 = jnp.maximum(acc + b1_ref[...], 0.0)
    s1_scr[...] += jnp.sum(h2, axis=0, keepdims=True)
    s2_scr[...] += jnp.sum(h2 * h2, axis=0, keepdims=True)
    ids = idx_ref[0, :, :]
    gi = jax.lax.broadcasted_iota(jnp.int32, (G, TR), 0)
    onehot = (gi == ids).astype(jnp.float32)
    pool_scr[...] += jnp.dot(onehot, h2, preferred_element_type=jnp.float32)
    cnt_scr[...] += onehot

    @pl.when(i == NSTEP - 1)
    def _():
        n_f = jnp.float32(N)
        mean1 = s1_scr[...] / n_f
        var1 = s2_scr[...] / n_f - mean1 * mean1
        a1 = g1_ref[...] / jnp.sqrt(var1 + 1e-5)
        c1 = be1_ref[...] - mean1 * a1
        cnt = jnp.sum(cnt_scr[...], axis=1, keepdims=True)
        pooled = pool_scr[...] * a1 + cnt * c1
        mean2 = jnp.mean(pooled, axis=0, keepdims=True)
        var2 = jnp.mean((pooled - mean2) ** 2, axis=0, keepdims=True)
        y = (pooled - mean2) / jnp.sqrt(var2 + 1e-5) * g2_ref[...] + be2_ref[...]
        out = jnp.dot(y, fw_ref[...], preferred_element_type=jnp.float32)
        out_ref[...] = jnp.maximum(out + fb_ref[...], 0.0)


def _const(shape):
    return pl.BlockSpec(shape, lambda i: tuple(0 for _ in shape))


@functools.partial(jax.jit, static_argnames=("interpret",))
def _run(adj, x, idx, W0, b0, W1, b1, gamma1, beta1, gamma2, beta2,
         fc1_W, fc1_b, interpret=False):
    f32 = jnp.float32
    row = pl.BlockSpec((TR, N), lambda i: (i, 0))
    outrow = pl.BlockSpec((TR, D), lambda i: (i, 0))
    f8 = jnp.float8_e4m3fn
    z1, mask = pl.pallas_call(
        _phase0_kernel,
        grid=(NSTEP,),
        in_specs=[row, _const((N, D)), _const((D, D)), _const((1, D)),
                  _const((D, D))],
        out_specs=[pl.BlockSpec((TR, 2 * D), lambda i: (i, 0)), row],
        out_shape=[jax.ShapeDtypeStruct((N, 2 * D), f8),
                   jax.ShapeDtypeStruct((N, N), f8)],
        interpret=interpret,
    )(adj, x, W0, b0.reshape(1, D), W1)

    idx_spec = pl.BlockSpec((1, 1, TR), lambda i: (i, 0, 0))
    out = pl.pallas_call(
        _phase1_kernel,
        grid=(NSTEP,),
        in_specs=[row, _const((N, 2 * D)), idx_spec,
                  _const((1, D)), _const((1, D)), _const((1, D)),
                  _const((1, D)), _const((1, D)), _const((D, D)),
                  _const((1, D))],
        out_specs=_const((G, D)),
        out_shape=jax.ShapeDtypeStruct((G, D), f32),
        scratch_shapes=[pltpu.VMEM((G, D), f32), pltpu.VMEM((G, TR), f32),
                        pltpu.VMEM((1, D), f32), pltpu.VMEM((1, D), f32)],
        interpret=interpret,
    )(mask, z1, idx.reshape(NSTEP, 1, TR).astype(jnp.int32),
      b1.reshape(1, D), gamma1.reshape(1, D), beta1.reshape(1, D),
      gamma2.reshape(1, D), beta2.reshape(1, D), fc1_W, fc1_b.reshape(1, D))
    return out


def kernel(adj, final_features, segment, idx, W0, b0, W1, b1,
           gamma1, beta1, gamma2, beta2, fc1_W, fc1_b):
    return _run(adj, final_features, idx, W0, b0, W1, b1,
                gamma1, beta1, gamma2, beta2, fc1_W, fc1_b)
